# baseline (speedup 1.0000x reference)
import functools

import jax
import jax.numpy as jnp
from jax import lax
from jax.experimental import pallas as pl
from jax.experimental.pallas import tpu as pltpu
from jax.experimental.pallas import tpu_sc as plsc

_BATCH = 4096
_N = 3
_EMBED_DIM = 64
_LANES = 16


def _sc_body(bpw, x_hbm, table_hbm, out_hbm,
             xv0, xv1, xv2, i0, i1, i2, r0, r1, r2, sem):
    wid = lax.axis_index("s") * 2 + lax.axis_index("c")
    base = wid * bpw

    pltpu.sync_copy(x_hbm.at[pl.ds(base, bpw)], xv0)
    pltpu.sync_copy(x_hbm.at[pl.ds(_BATCH + base, bpw)], xv1)
    pltpu.sync_copy(x_hbm.at[pl.ds(2 * _BATCH + base, bpw)], xv2)

    for c in range(bpw // _LANES):
        sl = pl.ds(c * _LANES, _LANES)
        g0 = xv0[sl]
        g1 = xv1[sl]
        g2 = xv2[sl]
        i0[sl] = g0
        i1[sl] = g0 * 64 + g1 + 64
        i2[sl] = g0 * 4096 + g1 * 64 + g2 + 4160

    cp0 = pltpu.async_copy(table_hbm.at[i0], r0, sem)
    cp1 = pltpu.async_copy(table_hbm.at[i1], r1, sem)
    cp2 = pltpu.async_copy(table_hbm.at[i2], r2, sem)
    cp0.wait()
    cp1.wait()
    cp2.wait()

    @pl.loop(0, bpw)
    def _(b):
        for k in range(_EMBED_DIM // _LANES):
            sl = pl.ds(k * _LANES, _LANES)
            r0[b, sl] = r0[b, sl] + r1[b, sl] + r2[b, sl]

    pltpu.sync_copy(r0, out_hbm.at[pl.ds(base, bpw)])


def kernel(x, embedding_weight):
    info = plsc.get_sparse_core_info()
    nw = info.num_cores * info.num_subcores
    bpw = _BATCH // nw
    mesh = plsc.VectorSubcoreMesh(core_axis_name="c", subcore_axis_name="s")

    sc_call = pl.kernel(
        functools.partial(_sc_body, bpw),
        out_type=jax.ShapeDtypeStruct((_BATCH, _EMBED_DIM), jnp.float32),
        mesh=mesh,
        compiler_params=pltpu.CompilerParams(use_tc_tiling_on_sc=False),
        scratch_types=[
            pltpu.VMEM((bpw,), jnp.int32),
            pltpu.VMEM((bpw,), jnp.int32),
            pltpu.VMEM((bpw,), jnp.int32),
            pltpu.VMEM((bpw,), jnp.int32),
            pltpu.VMEM((bpw,), jnp.int32),
            pltpu.VMEM((bpw,), jnp.int32),
            pltpu.VMEM((bpw, _EMBED_DIM), jnp.float32),
            pltpu.VMEM((bpw, _EMBED_DIM), jnp.float32),
            pltpu.VMEM((bpw, _EMBED_DIM), jnp.float32),
            pltpu.SemaphoreType.DMA,
        ],
    )
    return sc_call(x.T.reshape(-1), embedding_weight)


# speedup vs baseline: 1.5724x; 1.5724x over previous
import functools

import jax
import jax.numpy as jnp
from jax import lax
from jax.experimental import pallas as pl
from jax.experimental.pallas import tpu as pltpu
from jax.experimental.pallas import tpu_sc as plsc

_BATCH = 4096
_N = 3
_EMBED_DIM = 64
_LANES = 16


def _sc_body(bpw, x_hbm, table_hbm, out_hbm, xv0, xv1, xv2, rows, ov, sem):
    wid = lax.axis_index("s") * 2 + lax.axis_index("c")
    base = wid * bpw

    pltpu.sync_copy(x_hbm.at[pl.ds(base, bpw)], xv0)
    pltpu.sync_copy(x_hbm.at[pl.ds(_BATCH + base, bpw)], xv1)
    pltpu.sync_copy(x_hbm.at[pl.ds(2 * _BATCH + base, bpw)], xv2)

    @pl.loop(0, bpw, step=_LANES)
    def _(b):
        sl = pl.ds(b, _LANES)
        g0 = xv0[sl]
        g1 = xv1[sl]
        g2 = xv2[sl]
        n1 = g0 * 64 + g1 + 64
        n2 = g0 * 4096 + g1 * 64 + g2 + 4160
        for l in range(_LANES):
            pltpu.async_copy(table_hbm.at[pl.ds(g0[l], 1)],
                             rows.at[pl.ds(b + l, 1)], sem)
            pltpu.async_copy(table_hbm.at[pl.ds(n1[l], 1)],
                             rows.at[pl.ds(bpw + b + l, 1)], sem)
            pltpu.async_copy(table_hbm.at[pl.ds(n2[l], 1)],
                             rows.at[pl.ds(2 * bpw + b + l, 1)], sem)

    @pl.loop(0, _N * bpw)
    def _(j):
        pltpu.make_async_copy(table_hbm.at[pl.ds(0, 1)],
                              rows.at[pl.ds(j, 1)], sem).wait()

    @pl.loop(0, bpw)
    def _(b):
        for k in range(_EMBED_DIM // _LANES):
            sl = pl.ds(k * _LANES, _LANES)
            ov[b, sl] = (rows[b, sl] + rows[bpw + b, sl]
                         + rows[2 * bpw + b, sl])

    pltpu.sync_copy(ov, out_hbm.at[pl.ds(base, bpw)])


def kernel(x, embedding_weight):
    info = plsc.get_sparse_core_info()
    nw = info.num_cores * info.num_subcores
    bpw = _BATCH // nw
    mesh = plsc.VectorSubcoreMesh(core_axis_name="c", subcore_axis_name="s")

    sc_call = pl.kernel(
        functools.partial(_sc_body, bpw),
        out_type=jax.ShapeDtypeStruct((_BATCH, _EMBED_DIM), jnp.float32),
        mesh=mesh,
        scratch_types=[
            pltpu.VMEM((bpw,), jnp.int32),
            pltpu.VMEM((bpw,), jnp.int32),
            pltpu.VMEM((bpw,), jnp.int32),
            pltpu.VMEM((_N * bpw, _EMBED_DIM), jnp.float32),
            pltpu.VMEM((bpw, _EMBED_DIM), jnp.float32),
            pltpu.SemaphoreType.DMA,
        ],
    )
    return sc_call(x.T.reshape(-1), embedding_weight)
